# baseline (speedup 1.0000x reference)
import jax
import jax.numpy as jnp
from jax.experimental import pallas as pl
from jax.experimental.pallas import tpu as pltpu

_PROPS = 16
_NCHUNKS = 7


def _fps_body(src_ref, samples_ref, d_ref):
    c, hw = src_ref.shape[1], src_ref.shape[2]
    chunk = hw // _NCHUNKS

    acc = jnp.zeros((c,), jnp.float32)
    for k in range(_NCHUNKS):
        acc = acc + jnp.sum(src_ref[0, :, pl.ds(k * chunk, chunk)], axis=1)
    mean = acc / hw

    def dist_pass(s, reset):
        s2 = s[:, None]
        for k in range(_NCHUNKS):
            blk = src_ref[0, :, pl.ds(k * chunk, chunk)]
            nd = jnp.mean(jnp.abs(blk - s2), axis=0)
            if reset:
                d_ref[pl.ds(k * chunk, chunk)] = nd
            else:
                d_ref[pl.ds(k * chunk, chunk)] = jnp.minimum(
                    d_ref[pl.ds(k * chunk, chunk)], nd)

    def pick(i):
        idx = jnp.argmax(d_ref[...])
        g = idx // 128
        lane = idx - g * 128
        blk = src_ref[0, :, pl.ds(g * 128, 128)]
        lane_ids = jax.lax.broadcasted_iota(jnp.int32, (1, 128), 1)
        s = jnp.sum(jnp.where(lane_ids == lane, blk, 0.0), axis=1)
        samples_ref[0, pl.ds(i, 1), :] = s[None, :]
        return s

    dist_pass(mean, reset=True)
    s = pick(0)
    dist_pass(s, reset=True)

    def body(i, _):
        s = pick(i)
        dist_pass(s, reset=False)
        return 0

    jax.lax.fori_loop(1, _PROPS - 1, body, 0)
    pick(_PROPS - 1)


def kernel(inputs):
    b, h, w, c = inputs.shape
    hw = h * w
    src = inputs.reshape(b, hw, c)
    src_t = jnp.transpose(src, (0, 2, 1))
    samples = pl.pallas_call(
        _fps_body,
        grid=(b,),
        in_specs=[pl.BlockSpec((1, c, hw), lambda i: (i, 0, 0))],
        out_specs=pl.BlockSpec((1, _PROPS, c), lambda i: (i, 0, 0)),
        out_shape=jax.ShapeDtypeStruct((b, _PROPS, c), jnp.float32),
        scratch_shapes=[pltpu.VMEM((hw,), jnp.float32)],
        compiler_params=pltpu.CompilerParams(
            dimension_semantics=("parallel",)),
    )(src_t)
    return samples, src


# speedup vs baseline: 1.0688x; 1.0688x over previous
import jax
import jax.numpy as jnp
from jax.experimental import pallas as pl
from jax.experimental.pallas import tpu as pltpu

_PROPS = 16
_NCHUNKS = 28


def _fps_body(src_ref, samples_ref, d_ref):
    c, hw = src_ref.shape[1], src_ref.shape[2]
    chunk = hw // _NCHUNKS

    acc = jnp.zeros((c,), jnp.float32)
    for k in range(_NCHUNKS):
        acc = acc + jnp.sum(src_ref[0, :, pl.ds(k * chunk, chunk)], axis=1)
    mean = acc / hw

    def dist_pass(s, reset):
        s2 = s[:, None]
        for k in range(_NCHUNKS):
            blk = src_ref[0, :, pl.ds(k * chunk, chunk)]
            nd = jnp.mean(jnp.abs(blk - s2), axis=0)
            if reset:
                d_ref[pl.ds(k * chunk, chunk)] = nd
            else:
                d_ref[pl.ds(k * chunk, chunk)] = jnp.minimum(
                    d_ref[pl.ds(k * chunk, chunk)], nd)

    def pick(i):
        idx = jnp.argmax(d_ref[...])
        g = idx // 128
        lane = idx - g * 128
        blk = src_ref[0, :, pl.ds(g * 128, 128)]
        lane_ids = jax.lax.broadcasted_iota(jnp.int32, (1, 128), 1)
        s = jnp.sum(jnp.where(lane_ids == lane, blk, 0.0), axis=1)
        samples_ref[0, pl.ds(i, 1), :] = s[None, :]
        return s

    dist_pass(mean, reset=True)
    s = pick(0)
    dist_pass(s, reset=True)

    def body(i, _):
        s = pick(i)
        dist_pass(s, reset=False)
        return 0

    jax.lax.fori_loop(1, _PROPS - 1, body, 0)
    pick(_PROPS - 1)


def kernel(inputs):
    b, h, w, c = inputs.shape
    hw = h * w
    src = inputs.reshape(b, hw, c)
    src_t = jnp.transpose(src, (0, 2, 1))
    samples = pl.pallas_call(
        _fps_body,
        grid=(b,),
        in_specs=[pl.BlockSpec((1, c, hw), lambda i: (i, 0, 0))],
        out_specs=pl.BlockSpec((1, _PROPS, c), lambda i: (i, 0, 0)),
        out_shape=jax.ShapeDtypeStruct((b, _PROPS, c), jnp.float32),
        scratch_shapes=[pltpu.VMEM((hw,), jnp.float32)],
        compiler_params=pltpu.CompilerParams(
            dimension_semantics=("parallel",)),
    )(src_t)
    return samples, src


# speedup vs baseline: 1.1351x; 1.0620x over previous
import jax
import jax.numpy as jnp
from jax.experimental import pallas as pl
from jax.experimental.pallas import tpu as pltpu

_PROPS = 16
_NCHUNKS = 49


def _fps_body(src_ref, samples_ref, d_ref):
    c, hw = src_ref.shape[1], src_ref.shape[2]
    chunk = hw // _NCHUNKS

    acc = jnp.zeros((c,), jnp.float32)
    for k in range(_NCHUNKS):
        acc = acc + jnp.sum(src_ref[0, :, pl.ds(k * chunk, chunk)], axis=1)
    mean = acc / hw

    def dist_pass(s, reset):
        s2 = s[:, None]
        for k in range(_NCHUNKS):
            blk = src_ref[0, :, pl.ds(k * chunk, chunk)]
            nd = jnp.mean(jnp.abs(blk - s2), axis=0)
            if reset:
                d_ref[pl.ds(k * chunk, chunk)] = nd
            else:
                d_ref[pl.ds(k * chunk, chunk)] = jnp.minimum(
                    d_ref[pl.ds(k * chunk, chunk)], nd)

    def pick(i):
        idx = jnp.argmax(d_ref[...])
        g = idx // 128
        lane = idx - g * 128
        blk = src_ref[0, :, pl.ds(g * 128, 128)]
        lane_ids = jax.lax.broadcasted_iota(jnp.int32, (1, 128), 1)
        s = jnp.sum(jnp.where(lane_ids == lane, blk, 0.0), axis=1)
        samples_ref[0, pl.ds(i, 1), :] = s[None, :]
        return s

    dist_pass(mean, reset=True)
    s = pick(0)
    dist_pass(s, reset=True)

    def body(i, _):
        s = pick(i)
        dist_pass(s, reset=False)
        return 0

    jax.lax.fori_loop(1, _PROPS - 1, body, 0)
    pick(_PROPS - 1)


def kernel(inputs):
    b, h, w, c = inputs.shape
    hw = h * w
    src = inputs.reshape(b, hw, c)
    src_t = jnp.transpose(src, (0, 2, 1))
    samples = pl.pallas_call(
        _fps_body,
        grid=(b,),
        in_specs=[pl.BlockSpec((1, c, hw), lambda i: (i, 0, 0))],
        out_specs=pl.BlockSpec((1, _PROPS, c), lambda i: (i, 0, 0)),
        out_shape=jax.ShapeDtypeStruct((b, _PROPS, c), jnp.float32),
        scratch_shapes=[pltpu.VMEM((hw,), jnp.float32)],
        compiler_params=pltpu.CompilerParams(
            dimension_semantics=("parallel",)),
    )(src_t)
    return samples, src


# speedup vs baseline: 1.1682x; 1.0292x over previous
import jax
import jax.numpy as jnp
from jax.experimental import pallas as pl
from jax.experimental.pallas import tpu as pltpu

_PROPS = 16
_NCHUNKS = 98


def _fps_body(src_ref, samples_ref, d_ref):
    c, hw = src_ref.shape[1], src_ref.shape[2]
    chunk = hw // _NCHUNKS

    acc = jnp.zeros((c,), jnp.float32)
    for k in range(_NCHUNKS):
        acc = acc + jnp.sum(src_ref[0, :, pl.ds(k * chunk, chunk)], axis=1)
    mean = acc / hw

    def dist_pass(s, reset):
        s2 = s[:, None]
        for k in range(_NCHUNKS):
            blk = src_ref[0, :, pl.ds(k * chunk, chunk)]
            nd = jnp.mean(jnp.abs(blk - s2), axis=0)
            if reset:
                d_ref[pl.ds(k * chunk, chunk)] = nd
            else:
                d_ref[pl.ds(k * chunk, chunk)] = jnp.minimum(
                    d_ref[pl.ds(k * chunk, chunk)], nd)

    def pick(i):
        idx = jnp.argmax(d_ref[...])
        g = idx // 128
        lane = idx - g * 128
        blk = src_ref[0, :, pl.ds(g * 128, 128)]
        lane_ids = jax.lax.broadcasted_iota(jnp.int32, (1, 128), 1)
        s = jnp.sum(jnp.where(lane_ids == lane, blk, 0.0), axis=1)
        samples_ref[0, pl.ds(i, 1), :] = s[None, :]
        return s

    dist_pass(mean, reset=True)
    s = pick(0)
    dist_pass(s, reset=True)

    def body(i, _):
        s = pick(i)
        dist_pass(s, reset=False)
        return 0

    jax.lax.fori_loop(1, _PROPS - 1, body, 0)
    pick(_PROPS - 1)


def kernel(inputs):
    b, h, w, c = inputs.shape
    hw = h * w
    src = inputs.reshape(b, hw, c)
    src_t = jnp.transpose(src, (0, 2, 1))
    samples = pl.pallas_call(
        _fps_body,
        grid=(b,),
        in_specs=[pl.BlockSpec((1, c, hw), lambda i: (i, 0, 0))],
        out_specs=pl.BlockSpec((1, _PROPS, c), lambda i: (i, 0, 0)),
        out_shape=jax.ShapeDtypeStruct((b, _PROPS, c), jnp.float32),
        scratch_shapes=[pltpu.VMEM((hw,), jnp.float32)],
        compiler_params=pltpu.CompilerParams(
            dimension_semantics=("parallel",)),
    )(src_t)
    return samples, src


# speedup vs baseline: 1.2116x; 1.0372x over previous
import jax
import jax.numpy as jnp
from jax.experimental import pallas as pl
from jax.experimental.pallas import tpu as pltpu

_PROPS = 16
_NCHUNKS = 196


def _fps_body(src_ref, samples_ref, d_ref):
    c, hw = src_ref.shape[1], src_ref.shape[2]
    chunk = hw // _NCHUNKS

    acc = jnp.zeros((c,), jnp.float32)
    for k in range(_NCHUNKS):
        acc = acc + jnp.sum(src_ref[0, :, pl.ds(k * chunk, chunk)], axis=1)
    mean = acc / hw

    def dist_pass(s, reset):
        s2 = s[:, None]
        for k in range(_NCHUNKS):
            blk = src_ref[0, :, pl.ds(k * chunk, chunk)]
            nd = jnp.mean(jnp.abs(blk - s2), axis=0)
            if reset:
                d_ref[pl.ds(k * chunk, chunk)] = nd
            else:
                d_ref[pl.ds(k * chunk, chunk)] = jnp.minimum(
                    d_ref[pl.ds(k * chunk, chunk)], nd)

    def pick(i):
        idx = jnp.argmax(d_ref[...])
        g = idx // 128
        lane = idx - g * 128
        blk = src_ref[0, :, pl.ds(g * 128, 128)]
        lane_ids = jax.lax.broadcasted_iota(jnp.int32, (1, 128), 1)
        s = jnp.sum(jnp.where(lane_ids == lane, blk, 0.0), axis=1)
        samples_ref[0, pl.ds(i, 1), :] = s[None, :]
        return s

    dist_pass(mean, reset=True)
    s = pick(0)
    dist_pass(s, reset=True)

    def body(i, _):
        s = pick(i)
        dist_pass(s, reset=False)
        return 0

    jax.lax.fori_loop(1, _PROPS - 1, body, 0)
    pick(_PROPS - 1)


def kernel(inputs):
    b, h, w, c = inputs.shape
    hw = h * w
    src = inputs.reshape(b, hw, c)
    src_t = jnp.transpose(src, (0, 2, 1))
    samples = pl.pallas_call(
        _fps_body,
        grid=(b,),
        in_specs=[pl.BlockSpec((1, c, hw), lambda i: (i, 0, 0))],
        out_specs=pl.BlockSpec((1, _PROPS, c), lambda i: (i, 0, 0)),
        out_shape=jax.ShapeDtypeStruct((b, _PROPS, c), jnp.float32),
        scratch_shapes=[pltpu.VMEM((hw,), jnp.float32)],
        compiler_params=pltpu.CompilerParams(
            dimension_semantics=("parallel",)),
    )(src_t)
    return samples, src
